# baseline (speedup 1.0000x reference)
import functools

import jax
import jax.numpy as jnp
from jax import lax
from jax.experimental import pallas as pl
from jax.experimental.pallas import tpu as pltpu
from jax.experimental.pallas import tpu_sc as plsc

POOL = 10000
BATCH = 1024
SEL = 5
PLEN = 8
DIM = 768
ROW = PLEN * DIM
NROWS = BATCH * SEL
NW = 32
PER_W = NROWS // NW
CHUNK = 8
NCHUNKS = PER_W // CHUNK


@functools.partial(
    pl.kernel,
    mesh=plsc.VectorSubcoreMesh(core_axis_name="c", subcore_axis_name="s"),
    out_type=jax.ShapeDtypeStruct((NROWS, 3, ROW), jnp.float32),
    scratch_types=[
        pltpu.VMEM((NCHUNKS, CHUNK), jnp.int32),
        pltpu.VMEM((CHUNK, 1, ROW), jnp.float32),
        pltpu.SemaphoreType.DMA,
    ],
)
def _gather3(idx_hbm, a_hbm, b_hbm, c_hbm, out_hbm, idx_v, buf, sem):
    wid = lax.axis_index("s") * 2 + lax.axis_index("c")
    base = wid * PER_W
    pltpu.sync_copy(idx_hbm.at[wid], idx_v)
    for t, tab in enumerate((a_hbm, b_hbm, c_hbm)):
        def body(cidx, _, tab=tab, t=t):
            pltpu.async_copy(tab.at[idx_v.at[cidx]], buf, sem).wait()
            pltpu.sync_copy(buf, out_hbm.at[pl.ds(base + cidx * CHUNK, CHUNK), pl.ds(t, 1)])
            return ()
        lax.fori_loop(0, NCHUNKS, body, ())


def kernel(indices, part_A, part_B, part_C):
    idx = indices.reshape(NROWS).astype(jnp.int32).reshape(NW, NCHUNKS, CHUNK)
    a = part_A.reshape(POOL, 1, ROW)
    b = part_B.reshape(POOL, 1, ROW)
    c = part_C.reshape(POOL, 1, ROW)
    out = _gather3(idx, a, b, c)
    return out.reshape(BATCH, SEL, 3 * PLEN, DIM)


# speedup vs baseline: 1.0092x; 1.0092x over previous
import functools

import jax
import jax.numpy as jnp
from jax import lax
from jax.experimental import pallas as pl
from jax.experimental.pallas import tpu as pltpu
from jax.experimental.pallas import tpu_sc as plsc

POOL = 10000
BATCH = 1024
SEL = 5
PLEN = 8
DIM = 768
ROW = PLEN * DIM
NROWS = BATCH * SEL
NW = 32
PER_W = NROWS // NW
CHUNK = 4
NCHUNKS = PER_W // CHUNK
RING = 5
STEPS = NCHUNKS // RING


@functools.partial(
    pl.kernel,
    mesh=plsc.VectorSubcoreMesh(core_axis_name="c", subcore_axis_name="s"),
    out_type=jax.ShapeDtypeStruct((NROWS, 3, ROW), jnp.float32),
    scratch_types=[
        pltpu.VMEM((NCHUNKS, CHUNK), jnp.int32),
        pltpu.VMEM((RING, CHUNK, 1, ROW), jnp.float32),
        pltpu.SemaphoreType.DMA,
        pltpu.SemaphoreType.DMA,
    ],
)
def _gather3(idx_hbm, a_hbm, b_hbm, c_hbm, out_hbm, idx_v, bufs, gsem, wsem):
    wid = lax.axis_index("s") * 2 + lax.axis_index("c")
    base = wid * PER_W
    pltpu.sync_copy(idx_hbm.at[wid], idx_v)

    def out_slice(jb, t):
        return out_hbm.at[pl.ds(base + jb * CHUNK, CHUNK), pl.ds(t, 1)]

    for t, tab in enumerate((a_hbm, b_hbm, c_hbm)):
        def body(step, _, tab=tab, t=t):
            jb0 = step * RING

            @pl.when(step > 0)
            def _():
                for b in range(RING):
                    pltpu.make_async_copy(bufs.at[b], out_slice(0, t), wsem).wait()

            gathers = []
            for b in range(RING):
                gathers.append(
                    pltpu.async_copy(tab.at[idx_v.at[jb0 + b]], bufs.at[b], gsem)
                )
            for b in range(RING):
                gathers[b].wait()
            for b in range(RING):
                pltpu.async_copy(bufs.at[b], out_slice(jb0 + b, t), wsem)
            return ()

        lax.fori_loop(0, STEPS, body, ())
        for b in range(RING):
            pltpu.make_async_copy(bufs.at[b], out_slice(0, t), wsem).wait()


def kernel(indices, part_A, part_B, part_C):
    idx = indices.reshape(NROWS).astype(jnp.int32).reshape(NW, NCHUNKS, CHUNK)
    a = part_A.reshape(POOL, 1, ROW)
    b = part_B.reshape(POOL, 1, ROW)
    c = part_C.reshape(POOL, 1, ROW)
    out = _gather3(idx, a, b, c)
    return out.reshape(BATCH, SEL, 3 * PLEN, DIM)


# speedup vs baseline: 11.4169x; 11.3130x over previous
import functools

import jax
import jax.numpy as jnp
from jax import lax
from jax.experimental import pallas as pl
from jax.experimental.pallas import tpu as pltpu
from jax.experimental.pallas import tpu_sc as plsc

POOL = 10000
BATCH = 1024
SEL = 5
PLEN = 8
DIM = 768
NROWS = BATCH * SEL
NW = 32
PER_W = NROWS // NW
CHUNK = 4
NCHUNKS = PER_W // CHUNK
RING = 5
STEPS = NCHUNKS // RING


@functools.partial(
    pl.kernel,
    mesh=plsc.VectorSubcoreMesh(core_axis_name="c", subcore_axis_name="s"),
    out_type=jax.ShapeDtypeStruct((NROWS, 3 * PLEN, DIM), jnp.float32),
    scratch_types=[
        pltpu.VMEM((NCHUNKS, CHUNK), jnp.int32),
        pltpu.VMEM((RING, CHUNK, PLEN, DIM), jnp.float32),
        pltpu.SemaphoreType.DMA,
        pltpu.SemaphoreType.DMA,
    ],
)
def _gather3(idx_hbm, a_hbm, b_hbm, c_hbm, out_hbm, idx_v, bufs, gsem, wsem):
    wid = lax.axis_index("s") * 2 + lax.axis_index("c")
    base = wid * PER_W
    pltpu.sync_copy(idx_hbm.at[wid], idx_v)

    def out_slice(jb, t):
        return out_hbm.at[
            pl.ds(base + jb * CHUNK, CHUNK), pl.ds(t * PLEN, PLEN), :
        ]

    for t, tab in enumerate((a_hbm, b_hbm, c_hbm)):
        def body(step, _, tab=tab, t=t):
            jb0 = step * RING

            @pl.when(step > 0)
            def _():
                for b in range(RING):
                    pltpu.make_async_copy(bufs.at[b], out_slice(0, t), wsem).wait()

            gathers = []
            for b in range(RING):
                gathers.append(
                    pltpu.async_copy(tab.at[idx_v.at[jb0 + b]], bufs.at[b], gsem)
                )
            for b in range(RING):
                gathers[b].wait()
            for b in range(RING):
                pltpu.async_copy(bufs.at[b], out_slice(jb0 + b, t), wsem)
            return ()

        lax.fori_loop(0, STEPS, body, ())
        for b in range(RING):
            pltpu.make_async_copy(bufs.at[b], out_slice(0, t), wsem).wait()


def kernel(indices, part_A, part_B, part_C):
    idx = indices.reshape(NROWS).astype(jnp.int32).reshape(NW, NCHUNKS, CHUNK)
    out = _gather3(idx, part_A, part_B, part_C)
    return out.reshape(BATCH, SEL, 3 * PLEN, DIM)


# speedup vs baseline: 11.6587x; 1.0212x over previous
import functools

import jax
import jax.numpy as jnp
from jax import lax
from jax.experimental import pallas as pl
from jax.experimental.pallas import tpu as pltpu
from jax.experimental.pallas import tpu_sc as plsc

POOL = 10000
BATCH = 1024
SEL = 5
PLEN = 8
DIM = 768
NROWS = BATCH * SEL
NW = 32
PER_W = NROWS // NW
CHUNK = 4
NCHUNKS = PER_W // CHUNK
RING = 5
STEPS = NCHUNKS // RING


@functools.partial(
    pl.kernel,
    mesh=plsc.VectorSubcoreMesh(core_axis_name="c", subcore_axis_name="s"),
    out_type=jax.ShapeDtypeStruct((NROWS, 3 * PLEN, DIM), jnp.float32),
    scratch_types=[
        pltpu.VMEM((NCHUNKS, CHUNK), jnp.int32),
        pltpu.VMEM((RING, CHUNK, PLEN, DIM), jnp.float32),
        pltpu.SemaphoreType.DMA,
        pltpu.SemaphoreType.DMA,
    ],
)
def _gather3(idx_hbm, a_hbm, b_hbm, c_hbm, out_hbm, idx_v, bufs, gsem, wsem):
    wid = lax.axis_index("s") * 2 + lax.axis_index("c")
    base = wid * PER_W
    pltpu.sync_copy(idx_hbm.at[wid], idx_v)

    def out_slice(jb, t):
        return out_hbm.at[
            pl.ds(base + jb * CHUNK, CHUNK), pl.ds(t * PLEN, PLEN), :
        ]

    for t, tab in enumerate((a_hbm, b_hbm, c_hbm)):
        def body(step, _, tab=tab, t=t):
            jb0 = step * RING

            @pl.when(step > 0)
            def _():
                for b in range(RING):
                    pltpu.make_async_copy(bufs.at[b], out_slice(0, t), wsem).wait()

            gathers = []
            for b in range(RING):
                gathers.append(
                    pltpu.async_copy(tab.at[idx_v.at[jb0 + b]], bufs.at[b], gsem)
                )
            for b in range(RING):
                gathers[b].wait()
                pltpu.async_copy(bufs.at[b], out_slice(jb0 + b, t), wsem)
            return ()

        lax.fori_loop(0, STEPS, body, ())
        for b in range(RING):
            pltpu.make_async_copy(bufs.at[b], out_slice(0, t), wsem).wait()


def kernel(indices, part_A, part_B, part_C):
    idx = indices.reshape(NROWS).astype(jnp.int32).reshape(NW, NCHUNKS, CHUNK)
    out = _gather3(idx, part_A, part_B, part_C)
    return out.reshape(BATCH, SEL, 3 * PLEN, DIM)
